# baseline (speedup 1.0000x reference)
import functools

import jax
import jax.numpy as jnp
from jax import lax
from jax.experimental import pallas as pl
from jax.experimental.pallas import tpu as pltpu
from jax.experimental.pallas import tpu_sc as plsc

D = 64
NB, NS_SEQ = 16384, 50
NC, NS = 2, 16
NW = NC * NS
N_SUB = 4
S_PIECES = 2


def _make_kernel(ns_p):
  sub_b = 1
  while 2 * sub_b * ns_p <= 128:
    sub_b *= 2
  sub = sub_b * ns_p
  chunk = sub * N_SUB
  b_chunk = N_SUB * sub_b
  bp = NB * ns_p
  b_per_w = bp // NW
  idx_rows = b_per_w // sub
  n_chunks = b_per_w // chunk
  assert b_per_w % chunk == 0 and n_chunks % 2 == 0
  t_iters = n_chunks // 2

  mesh = plsc.VectorSubcoreMesh(core_axis_name="c", subcore_axis_name="s")

  @functools.partial(
      pl.kernel,
      mesh=mesh,
      compiler_params=pltpu.CompilerParams(use_tc_tiling_on_sc=False),
      out_type=jax.ShapeDtypeStruct((NB, ns_p, D), jnp.float32),
      scratch_types=[
          pltpu.VMEM((idx_rows, sub), jnp.int32),
          pltpu.VMEM((2, N_SUB, sub, D), jnp.float32),
          pltpu.SemaphoreType.DMA,
          pltpu.SemaphoreType.DMA,
          pltpu.SemaphoreType.DMA,
          pltpu.SemaphoreType.DMA,
      ],
  )
  def k(idx_hbm, table_hbm, out_hbm, idx_v, rows_v, sg0, sg1, sw0, sw1):
    wid = lax.axis_index("s") * NC + lax.axis_index("c")
    b_base = wid * (b_per_w // ns_p)
    pltpu.sync_copy(idx_hbm.at[pl.ds(wid * idx_rows, idx_rows)], idx_v)
    sg = (sg0, sg1)
    sw = (sw0, sw1)

    def fire(buf, c):
      for j in range(N_SUB):
        pltpu.async_copy(table_hbm.at[idx_v.at[c * N_SUB + j]],
                         rows_v.at[buf, j], sg[buf])

    def wait_gathers(buf):
      for j in range(N_SUB):
        pltpu.make_async_copy(table_hbm.at[idx_v.at[j]],
                              rows_v.at[buf, j], sg[buf]).wait()

    def write(buf, c):
      b0 = b_base + c * b_chunk
      for j in range(N_SUB):
        for kk in range(sub_b):
          pltpu.async_copy(
              rows_v.at[buf, j, pl.ds(kk * ns_p, ns_p)],
              out_hbm.at[b0 + j * sub_b + kk], sw[buf])

    def wait_write(buf):
      for j in range(N_SUB):
        for kk in range(sub_b):
          pltpu.make_async_copy(
              rows_v.at[buf, j, pl.ds(kk * ns_p, ns_p)],
              out_hbm.at[b_base + j * sub_b + kk], sw[buf]).wait()

    fire(0, 0)
    fire(1, 1)

    def body(t, carry):
      c0 = 2 * t
      c1 = c0 + 1
      wait_gathers(0)
      write(0, c0)
      wait_gathers(1)
      write(1, c1)
      wait_write(0)

      @pl.when(t < t_iters - 1)
      def _():
        fire(0, c0 + 2)

      wait_write(1)

      @pl.when(t < t_iters - 1)
      def _():
        fire(1, c1 + 2)

      return carry

    lax.fori_loop(0, t_iters, body, 0)

  return k, sub


_NS_P = NS_SEQ // S_PIECES
_gather_call, _SUB = _make_kernel(_NS_P)


@jax.jit
def kernel(inputs, weight):
  wlin = jnp.pad(weight, ((0, 0), (0, D))).reshape(2 * weight.shape[0], D)
  idx2 = inputs.astype(jnp.int32) * 2
  outs = []
  for p in range(S_PIECES):
    sl = idx2[:, p * _NS_P:(p + 1) * _NS_P]
    idx = sl.reshape(-1).reshape(NB * _NS_P // _SUB, _SUB)
    outs.append(_gather_call(idx, wlin))
  return jnp.concatenate(outs, axis=1)


# speedup vs baseline: 1.0885x; 1.0885x over previous
import functools

import jax
import jax.numpy as jnp
from jax import lax
from jax.experimental import pallas as pl
from jax.experimental.pallas import tpu as pltpu
from jax.experimental.pallas import tpu_sc as plsc

D = 64
NB, NS_SEQ = 16384, 50
NC, NS = 2, 16
NW = NC * NS
N_SUB = 4
S_PIECES = 1


def _make_kernel(ns_p):
  sub_b = 1
  while 2 * sub_b * ns_p <= 128:
    sub_b *= 2
  sub = sub_b * ns_p
  chunk = sub * N_SUB
  b_chunk = N_SUB * sub_b
  bp = NB * ns_p
  b_per_w = bp // NW
  idx_rows = b_per_w // sub
  n_chunks = b_per_w // chunk
  assert b_per_w % chunk == 0 and n_chunks % 2 == 0
  t_iters = n_chunks // 2

  mesh = plsc.VectorSubcoreMesh(core_axis_name="c", subcore_axis_name="s")

  @functools.partial(
      pl.kernel,
      mesh=mesh,
      compiler_params=pltpu.CompilerParams(use_tc_tiling_on_sc=False),
      out_type=jax.ShapeDtypeStruct((NB, ns_p, D), jnp.float32),
      scratch_types=[
          pltpu.VMEM((idx_rows, sub), jnp.int32),
          pltpu.VMEM((2, N_SUB, sub, D), jnp.float32),
          pltpu.SemaphoreType.DMA,
          pltpu.SemaphoreType.DMA,
          pltpu.SemaphoreType.DMA,
          pltpu.SemaphoreType.DMA,
      ],
  )
  def k(idx_hbm, table_hbm, out_hbm, idx_v, rows_v, sg0, sg1, sw0, sw1):
    wid = lax.axis_index("s") * NC + lax.axis_index("c")
    b_base = wid * (b_per_w // ns_p)
    pltpu.sync_copy(idx_hbm.at[pl.ds(wid * idx_rows, idx_rows)], idx_v)
    sg = (sg0, sg1)
    sw = (sw0, sw1)

    def fire(buf, c):
      for j in range(N_SUB):
        pltpu.async_copy(table_hbm.at[idx_v.at[c * N_SUB + j]],
                         rows_v.at[buf, j], sg[buf])

    def wait_gathers(buf):
      for j in range(N_SUB):
        pltpu.make_async_copy(table_hbm.at[idx_v.at[j]],
                              rows_v.at[buf, j], sg[buf]).wait()

    def write(buf, c):
      b0 = b_base + c * b_chunk
      for j in range(N_SUB):
        for kk in range(sub_b):
          pltpu.async_copy(
              rows_v.at[buf, j, pl.ds(kk * ns_p, ns_p)],
              out_hbm.at[b0 + j * sub_b + kk], sw[buf])

    def wait_write(buf):
      for j in range(N_SUB):
        for kk in range(sub_b):
          pltpu.make_async_copy(
              rows_v.at[buf, j, pl.ds(kk * ns_p, ns_p)],
              out_hbm.at[b_base + j * sub_b + kk], sw[buf]).wait()

    fire(0, 0)
    fire(1, 1)

    def body(t, carry):
      c0 = 2 * t
      c1 = c0 + 1
      wait_gathers(0)
      write(0, c0)
      wait_gathers(1)
      write(1, c1)
      wait_write(0)

      @pl.when(t < t_iters - 1)
      def _():
        fire(0, c0 + 2)

      wait_write(1)

      @pl.when(t < t_iters - 1)
      def _():
        fire(1, c1 + 2)

      return carry

    lax.fori_loop(0, t_iters, body, 0)

  return k, sub


_NS_P = NS_SEQ // S_PIECES
_gather_call, _SUB = _make_kernel(_NS_P)


@jax.jit
def kernel(inputs, weight):
  wlin = jnp.pad(weight, ((0, 0), (0, D))).reshape(2 * weight.shape[0], D)
  idx2 = inputs.astype(jnp.int32) * 2
  outs = []
  for p in range(S_PIECES):
    sl = idx2[:, p * _NS_P:(p + 1) * _NS_P]
    idx = sl.reshape(-1).reshape(NB * _NS_P // _SUB, _SUB)
    outs.append(_gather_call(idx, wlin))
  return jnp.concatenate(outs, axis=1)
